# baseline (speedup 1.0000x reference)
import functools

import jax
import jax.numpy as jnp
from jax import lax
from jax.experimental import pallas as pl
from jax.experimental.pallas import tpu as pltpu
from jax.experimental.pallas import tpu_sc as plsc

VOCAB = 1000000
EMBED = 64
BATCH = 4096
SEQ = 200

NC = 2
NS = 16
NW = NC * NS

B_TOTAL = BATCH * SEQ
B_PER_W = B_TOTAL // NW
CHUNK = 128
NCHUNKS = B_PER_W // CHUNK
NBUF = 2
NOUTER = NCHUNKS // NBUF


def _gather_kernel(table_hbm, idx_hbm, out_hbm, idx_v, r0, r1, s0, s1):
    rows = (r0, r1)
    sems = (s0, s1)

    wid = lax.axis_index("s") * NC + lax.axis_index("c")
    base = pl.multiple_of(wid * B_PER_W, B_PER_W)

    pltpu.sync_copy(idx_hbm.at[wid], idx_v)

    def gather_start(c, b):
        pltpu.async_copy(table_hbm.at[idx_v.at[c]], rows[b], sems[b])

    def gather_wait(c, b):
        pltpu.make_async_copy(table_hbm.at[idx_v.at[c]], rows[b], sems[b]).wait()

    def store_out(c, b):
        off = pl.multiple_of(base + c * CHUNK, CHUNK)
        pltpu.sync_copy(rows[b], out_hbm.at[pl.ds(off, CHUNK)])

    for b in range(NBUF):
        gather_start(b, b)

    def outer(i, carry):
        for b in range(NBUF):
            c = i * NBUF + b
            gather_wait(c, b)
            store_out(c, b)
            gather_start(c + NBUF, b)
        return carry

    lax.fori_loop(0, NOUTER - 1, outer, 0, unroll=False)

    for b in range(NBUF):
        c = (NOUTER - 1) * NBUF + b
        gather_wait(c, b)
        store_out(c, b)


@jax.jit
def _embedding_lookup(sequence, table):
    idx = sequence.reshape(NW, NCHUNKS, CHUNK).astype(jnp.int32)

    mesh = plsc.VectorSubcoreMesh(core_axis_name="c", subcore_axis_name="s")
    out = pl.kernel(
        _gather_kernel,
        out_type=jax.ShapeDtypeStruct((B_TOTAL, EMBED), jnp.float32),
        mesh=mesh,
        scratch_types=[
            pltpu.VMEM((NCHUNKS, CHUNK), jnp.int32),
            pltpu.VMEM((CHUNK, EMBED), jnp.float32),
            pltpu.VMEM((CHUNK, EMBED), jnp.float32),
            pltpu.SemaphoreType.DMA,
            pltpu.SemaphoreType.DMA,
        ],
        compiler_params=pltpu.CompilerParams(use_tc_tiling_on_sc=False),
    )(table, idx)
    return out.reshape(BATCH, SEQ, EMBED)


def kernel(sequence, table):
    return _embedding_lookup(sequence, table)


# speedup vs baseline: 1.0189x; 1.0189x over previous
import jax
import jax.numpy as jnp
from jax import lax
from jax.experimental import pallas as pl
from jax.experimental.pallas import tpu as pltpu
from jax.experimental.pallas import tpu_sc as plsc

VOCAB = 1000000
EMBED = 64
BATCH = 4096
SEQ = 200

NC = 2
NS = 16
NW = NC * NS

B_TOTAL = BATCH * SEQ
B_PER_W = B_TOTAL // NW
CHUNK = 128
NCHUNKS = B_PER_W // CHUNK
NBUF = 8
NGROUPS = NCHUNKS // NBUF


def _gather_kernel(table_hbm, idx_hbm, out_hbm, idx_v, rows_v, gsem, osem):
    wid = lax.axis_index("s") * NC + lax.axis_index("c")
    base = pl.multiple_of(wid * B_PER_W, B_PER_W)

    pltpu.sync_copy(idx_hbm.at[wid], idx_v)

    def gather_start(c, b):
        pltpu.async_copy(table_hbm.at[idx_v.at[c]], rows_v.at[b], gsem.at[b])

    def gather_wait(c, b):
        pltpu.make_async_copy(
            table_hbm.at[idx_v.at[c]], rows_v.at[b], gsem.at[b]
        ).wait()

    def out_slice(c):
        return out_hbm.at[pl.ds(pl.multiple_of(base + c * CHUNK, CHUNK), CHUNK)]

    def store_start(c, b):
        pltpu.async_copy(rows_v.at[b], out_slice(c), osem.at[b])

    def store_wait(c, b):
        pltpu.make_async_copy(rows_v.at[b], out_slice(c), osem.at[b]).wait()

    def step(c, b, first, last):
        gather_wait(c, b)
        store_start(c, b)
        nb = (b + NBUF - 1) % NBUF
        if not last:
            if not first:
                store_wait(c - 1, nb)
            gather_start(c + NBUF - 1, nb)

    for b in range(NBUF - 1):
        gather_start(b, b)

    for b in range(NBUF):
        step(b, b, first=(b == 0), last=False)

    def group(g, carry):
        for b in range(NBUF):
            step(g * NBUF + b, b, first=False, last=False)
        return carry

    lax.fori_loop(1, NGROUPS - 1, group, 0, unroll=False)

    for b in range(NBUF):
        c = (NGROUPS - 1) * NBUF + b
        step(c, b, first=False, last=(b != 0))

    for b in range(NBUF):
        store_wait(NCHUNKS - NBUF + b, b)


@jax.jit
def _embedding_lookup(sequence, table):
    idx = sequence.reshape(NW, NCHUNKS, CHUNK).astype(jnp.int32)

    mesh = plsc.VectorSubcoreMesh(core_axis_name="c", subcore_axis_name="s")
    out = pl.kernel(
        _gather_kernel,
        out_type=jax.ShapeDtypeStruct((B_TOTAL, EMBED), jnp.float32),
        mesh=mesh,
        scratch_types=[
            pltpu.VMEM((NCHUNKS, CHUNK), jnp.int32),
            pltpu.VMEM((NBUF, CHUNK, EMBED), jnp.float32),
            pltpu.SemaphoreType.DMA((NBUF,)),
            pltpu.SemaphoreType.DMA((NBUF,)),
        ],
        compiler_params=pltpu.CompilerParams(use_tc_tiling_on_sc=False),
    )(table, idx)
    return out.reshape(BATCH, SEQ, EMBED)


def kernel(sequence, table):
    return _embedding_lookup(sequence, table)


# speedup vs baseline: 1.0205x; 1.0016x over previous
import jax
import jax.numpy as jnp
from jax import lax
from jax.experimental import pallas as pl
from jax.experimental.pallas import tpu as pltpu
from jax.experimental.pallas import tpu_sc as plsc

VOCAB = 1000000
EMBED = 64
BATCH = 4096
SEQ = 200

NC = 2
NS = 16
NW = NC * NS

B_TOTAL = BATCH * SEQ
B_PER_W = B_TOTAL // NW
CHUNK = 128
NCHUNKS = B_PER_W // CHUNK
NBUF = 8
NGROUPS = NCHUNKS // NBUF


def _gather_kernel(table_hbm, idx_hbm, out_hbm, idx_v, rows_v, gsem, osem):
    wid = lax.axis_index("s") * NC + lax.axis_index("c")
    base = pl.multiple_of(wid * B_PER_W, B_PER_W)

    pltpu.sync_copy(idx_hbm.at[wid], idx_v)

    def gather_start(c, b):
        pltpu.async_copy(table_hbm.at[idx_v.at[c]], rows_v.at[b], gsem.at[b])

    def gather_wait(c, b):
        pltpu.make_async_copy(
            table_hbm.at[idx_v.at[c]], rows_v.at[b], gsem.at[b]
        ).wait()

    def out_slice(c):
        return out_hbm.at[pl.ds(pl.multiple_of(base + c * CHUNK, CHUNK), CHUNK)]

    def store_start(c, b):
        pltpu.async_copy(rows_v.at[b], out_slice(c), osem.at[b])

    def store_wait(c, b):
        pltpu.make_async_copy(rows_v.at[b], out_slice(c), osem.at[b]).wait()

    def step(c, b, first, last):
        gather_wait(c, b)
        store_start(c, b)
        nb = (b + NBUF - 1) % NBUF
        if not last:
            if not first:
                store_wait(c - 1, nb)
            gather_start(c + NBUF - 1, nb)

    for b in range(NBUF - 1):
        gather_start(b, b)

    for b in range(NBUF):
        step(b, b, first=(b == 0), last=False)

    def group(g, carry):
        for b in range(NBUF):
            step(g * NBUF + b, b, first=False, last=False)
        return carry

    lax.fori_loop(1, NGROUPS - 1, group, 0, unroll=False)

    for b in range(NBUF):
        c = (NGROUPS - 1) * NBUF + b
        step(c, b, first=False, last=(b != 0))

    for b in range(NBUF):
        store_wait(NCHUNKS - NBUF + b, b)


@jax.jit
def _embedding_lookup(sequence, table):
    idx = sequence.reshape(NW, NCHUNKS, CHUNK).astype(jnp.int32)

    mesh = plsc.VectorSubcoreMesh(core_axis_name="c", subcore_axis_name="s")
    out = pl.kernel(
        _gather_kernel,
        out_type=jax.ShapeDtypeStruct((B_TOTAL, EMBED), jnp.float32),
        mesh=mesh,
        scratch_types=[
            pltpu.VMEM((NCHUNKS, CHUNK), jnp.int32),
            pltpu.VMEM((NBUF, CHUNK, EMBED), jnp.float32),
            pltpu.SemaphoreType.DMA((NBUF,)),
            pltpu.SemaphoreType.DMA((NBUF,)),
        ],
        compiler_params=pltpu.CompilerParams(
            use_tc_tiling_on_sc=False, skip_device_barrier=True
        ),
    )(table, idx)
    return out.reshape(BATCH, SEQ, EMBED)


def kernel(sequence, table):
    return _embedding_lookup(sequence, table)


# speedup vs baseline: 1.2468x; 1.2217x over previous
import jax
import jax.numpy as jnp
from jax import lax
from jax.experimental import pallas as pl
from jax.experimental.pallas import tpu as pltpu
from jax.experimental.pallas import tpu_sc as plsc

VOCAB = 1000000
EMBED = 64
BATCH = 4096
SEQ = 200

NC = 2
NS = 16
NW = NC * NS

B_TOTAL = BATCH * SEQ
B_PER_W = B_TOTAL // NW
CHUNK = 128
NCHUNKS = B_PER_W // CHUNK
NBUF = 4
NGROUPS = NCHUNKS // NBUF

PAD = 2 * EMBED


def _gather_kernel(table_hbm, idx_hbm, out_hbm, idx_v, rows_v, gsem, osem):
    wid = lax.axis_index("s") * NC + lax.axis_index("c")
    base = pl.multiple_of(wid * B_PER_W, B_PER_W)

    pltpu.sync_copy(idx_hbm.at[wid], idx_v)

    def gather_start(c, b):
        pltpu.async_copy(table_hbm.at[idx_v.at[c]], rows_v.at[b], gsem.at[b])

    def gather_wait(c, b):
        pltpu.make_async_copy(
            table_hbm.at[idx_v.at[c]], rows_v.at[b], gsem.at[b]
        ).wait()

    def out_slice(c):
        return out_hbm.at[pl.ds(pl.multiple_of(base + c * CHUNK, CHUNK), CHUNK)]

    def store_start(c, b):
        pltpu.async_copy(rows_v.at[b], out_slice(c), osem.at[b])

    def store_wait(c, b):
        pltpu.make_async_copy(rows_v.at[b], out_slice(c), osem.at[b]).wait()

    def step(c, b, first, last):
        gather_wait(c, b)
        store_start(c, b)
        nb = (b + NBUF - 1) % NBUF
        if not last:
            if not first:
                store_wait(c - 1, nb)
            gather_start(c + NBUF - 1, nb)

    for b in range(NBUF - 1):
        gather_start(b, b)

    for b in range(NBUF):
        step(b, b, first=(b == 0), last=False)

    def group(g, carry):
        for b in range(NBUF):
            step(g * NBUF + b, b, first=False, last=False)
        return carry

    lax.fori_loop(1, NGROUPS - 1, group, 0, unroll=False)

    for b in range(NBUF):
        c = (NGROUPS - 1) * NBUF + b
        step(c, b, first=False, last=(b != 0))

    for b in range(NBUF):
        store_wait(NCHUNKS - NBUF + b, b)


@jax.jit
def _embedding_lookup(sequence, table):
    idx = sequence.reshape(NW, NCHUNKS, CHUNK).astype(jnp.int32)
    table_p = jnp.pad(table, ((0, 0), (0, PAD - EMBED)))

    mesh = plsc.VectorSubcoreMesh(core_axis_name="c", subcore_axis_name="s")
    out = pl.kernel(
        _gather_kernel,
        out_type=jax.ShapeDtypeStruct((B_TOTAL, PAD), jnp.float32),
        mesh=mesh,
        scratch_types=[
            pltpu.VMEM((NCHUNKS, CHUNK), jnp.int32),
            pltpu.VMEM((NBUF, CHUNK, PAD), jnp.float32),
            pltpu.SemaphoreType.DMA((NBUF,)),
            pltpu.SemaphoreType.DMA((NBUF,)),
        ],
        compiler_params=pltpu.CompilerParams(use_tc_tiling_on_sc=True),
    )(table_p, idx)
    return out[:, :EMBED].reshape(BATCH, SEQ, EMBED)


def kernel(sequence, table):
    return _embedding_lookup(sequence, table)
